# baseline (speedup 1.0000x reference)
import functools

import jax
import jax.numpy as jnp
from jax import lax
from jax.experimental import pallas as pl
from jax.experimental.pallas import tpu as pltpu

N_SEG = 1024
WIN = 128
N_WIN = N_SEG // WIN
ROWS = 232
NEG = -1e30


def _pick_block(n):
    for k in (2560, 2000, 1600, 1280, 1000, 800, 640, 512, 320, 256, 128, 64, 32, 16, 8):
        if n % k == 0:
            return k
    return n


def _body(nb, k_pts,
          s_ref, v_ref, pos_ref, bid_ref,
          te_ref,
          a1v1_ref, a1v2_ref, a1swv_ref, a1sws_ref, a1gwt_ref, a1gb_ref, a1d_ref,
          a2v1_ref, a2swv_ref, a2sws_ref,
          n1v1_ref, n1v2_ref, n1swv_ref, n1sws_ref, n1gwt_ref, n1gb_ref, n1d_ref,
          n2v1_ref, n2v2_ref, n2swv_ref, n2sws_ref, n2gwt_ref, n2gb_ref,
          smat_ref, emat_ref,
          out_ref, m_ref, den_ref):
    i = pl.program_id(0)

    @pl.when(i == 0)
    def _init():
        out_ref[:] = jnp.zeros_like(out_ref)
        m_ref[:] = jnp.full_like(m_ref, NEG)
        den_ref[:] = jnp.zeros_like(den_ref)

    f32 = jnp.float32
    bf16 = jnp.bfloat16
    dot = functools.partial(jnp.dot, preferred_element_type=f32)

    def dot16(a, b):
        return jnp.dot(a.astype(bf16), b.astype(bf16),
                       preferred_element_type=f32)
    S = smat_ref[:]
    E = emat_ref[:]

    s0 = s_ref[:] + te_ref[:]
    v0 = v_ref[:]

    def gv(s, v, A1, A2, sWv, sWs, gWt, gb):
        vi = dot16(v, A1)
        vn = jnp.sqrt(dot(vi * vi, S))
        os_ = dot16(vn, sWv) + dot16(s, sWs)
        ov = dot16(vi, A2)
        gate = jax.nn.sigmoid(dot16(os_, gWt) + gb)
        return os_, dot(gate, E) * ov

    def vlrelu(x, D):
        d = dot16(x, D)
        dt = dot(x * d, S)
        dsq = dot(d * d, S)
        coef = jnp.where(dt >= 0.0, 0.0, dt / (dsq + 1e-9))
        return 0.2 * x + 0.8 * (x - dot(coef, E) * d)

    def lrelu(x):
        return jnp.where(x >= 0.0, x, 0.01 * x)

    sa, va = gv(s0, v0, a1v1_ref[:], a1v2_ref[:], a1swv_ref[:], a1sws_ref[:],
                a1gwt_ref[:], a1gb_ref[:])
    va = vlrelu(va, a1d_ref[:])
    sa = lrelu(sa)
    vi2 = dot16(va, a2v1_ref[:])
    vn2 = jnp.sqrt(dot(vi2 * vi2, S))
    att_b = dot(vn2, a2swv_ref[:]) + dot(sa, a2sws_ref[:])

    sn, vn_ = gv(s0, v0, n1v1_ref[:], n1v2_ref[:], n1swv_ref[:], n1sws_ref[:],
                 n1gwt_ref[:], n1gb_ref[:])
    vn_ = vlrelu(vn_, n1d_ref[:])
    sn = lrelu(sn)
    hs, hv = gv(sn, vn_, n2v1_ref[:], n2v2_ref[:], n2swv_ref[:], n2sws_ref[:],
                n2gwt_ref[:], n2gb_ref[:])

    pos8 = jnp.concatenate(
        [pos_ref[:], jnp.zeros((k_pts, 5), dtype=f32)], axis=1)

    w2 = jnp.concatenate([jnp.full((1, WIN), 32.0, dtype=f32),
                          jnp.ones((1, WIN), dtype=f32)], axis=0)
    bid_b = dot(bid_ref[:], w2)
    bid_lo = bid_ref[0, 0] * 32.0 + bid_ref[0, 1]
    bid_hi = bid_ref[k_pts - 1, 0] * 32.0 + bid_ref[k_pts - 1, 1]
    lane = lax.broadcasted_iota(jnp.int32, (1, WIN), 1).astype(f32)

    for w in range(N_WIN):
        base = w * WIN

        @pl.when((bid_hi >= base) & (bid_lo < base + WIN))
        def _win(w=w, base=base):
            O = bid_b == (float(base) + lane)
            att_m = jnp.where(O, att_b, NEG)
            m_old = m_ref[w:w + 1, :]
            m_new = jnp.maximum(m_old, jnp.max(att_m, axis=0, keepdims=True))
            scale = jnp.exp(m_old - m_new)
            p = jnp.where(O, jnp.exp(att_m - m_new), 0.0)
            m_ref[w:w + 1, :] = m_new
            den_ref[w:w + 1, :] = (den_ref[w:w + 1, :] * scale
                                   + jnp.sum(p, axis=0, keepdims=True))
            cn = (((0,), (0,)), ((), ()))
            part_hs = lax.dot_general(hs, p, cn, preferred_element_type=f32)
            part_hv = lax.dot_general(hv, p, cn, preferred_element_type=f32)
            part_po = lax.dot_general(pos8, p, cn, preferred_element_type=f32)
            sl = slice(base, base + WIN)
            out_ref[0:128, sl] = out_ref[0:128, sl] * scale + part_hs
            out_ref[128:224, sl] = out_ref[128:224, sl] * scale + part_hv
            out_ref[224:232, sl] = out_ref[224:232, sl] * scale + part_po

    @pl.when(i == nb - 1)
    def _fin():
        den = den_ref[:]
        den_safe = jnp.where(den == 0.0, 1.0, den)
        for w in range(N_WIN):
            sl = slice(w * WIN, (w + 1) * WIN)
            out_ref[:, sl] = out_ref[:, sl] / den_safe[w:w + 1, :]


def kernel(h_att_sca, h_att_vec, pos_context, batch_id, t, params):
    n = h_att_sca.shape[0]
    hv_ch = h_att_vec.shape[1]
    k_pts = _pick_block(n)
    nb = n // k_pts
    f32 = jnp.float32

    eye3 = jnp.eye(3, dtype=f32)
    ones31 = jnp.ones((3, 1), dtype=f32)

    def kron3(W):
        return jnp.kron(W.T, eye3)

    p = params
    smat = jnp.kron(jnp.eye(hv_ch, dtype=f32), ones31)
    emat = smat.T

    te = p['time_embed'][t][None, :]
    v_flat = h_att_vec.reshape(n, -1)
    bidi = batch_id.astype(jnp.int32)
    bid2 = jnp.stack([(bidi >> 5).astype(f32),
                      (bidi & 31).astype(f32)], axis=1)

    hv_n = 3 * hv_ch
    args = [
        h_att_sca, v_flat, pos_context, bid2,
        te,
        kron3(p['a1_vW1']), kron3(p['a1_vW2']),
        p['a1_sW'][:, :hv_ch].T, p['a1_sW'][:, hv_ch:].T,
        p['a1_gW'].T, p['a1_gb'][None, :], kron3(p['a1_dW']),
        kron3(p['a2_vW1']),
        jnp.tile(p['a2_sW'][:, :hv_ch].T, (1, WIN)),
        jnp.tile(p['a2_sW'][:, hv_ch:].T, (1, WIN)),
        kron3(p['n1_vW1']), kron3(p['n1_vW2']),
        p['n1_sW'][:, :hv_ch].T, p['n1_sW'][:, hv_ch:].T,
        p['n1_gW'].T, p['n1_gb'][None, :], kron3(p['n1_dW']),
        kron3(p['n2_vW1']), kron3(p['n2_vW2']),
        p['n2_sW'][:, :hv_ch].T, p['n2_sW'][:, hv_ch:].T,
        p['n2_gW'].T, p['n2_gb'][None, :],
        smat, emat,
    ]

    def fixed(a):
        shape = a.shape
        return pl.BlockSpec(shape, lambda i: (0,) * len(shape))

    in_specs = [
        pl.BlockSpec((k_pts, 128), lambda i: (i, 0)),
        pl.BlockSpec((k_pts, hv_n), lambda i: (i, 0)),
        pl.BlockSpec((k_pts, 3), lambda i: (i, 0)),
        pl.BlockSpec((k_pts, 2), lambda i: (i, 0)),
    ] + [fixed(a) for a in args[4:]]

    out = pl.pallas_call(
        functools.partial(_body, nb, k_pts),
        grid=(nb,),
        in_specs=in_specs,
        out_specs=pl.BlockSpec((ROWS, N_SEG), lambda i: (0, 0)),
        out_shape=jax.ShapeDtypeStruct((ROWS, N_SEG), f32),
        scratch_shapes=[
            pltpu.VMEM((N_WIN, WIN), f32),
            pltpu.VMEM((N_WIN, WIN), f32),
        ],
        compiler_params=pltpu.CompilerParams(
            dimension_semantics=("arbitrary",)),
    )(*args)

    feat = out[0:128, :].T
    vec = out[128:128 + hv_n, :].T.reshape(N_SEG, hv_ch, 3)
    pos = out[224:227, :].T
    return feat, vec, pos


# speedup vs baseline: 1.0105x; 1.0105x over previous
import functools

import jax
import jax.numpy as jnp
from jax import lax
from jax.experimental import pallas as pl
from jax.experimental.pallas import tpu as pltpu

N_SEG = 1024
WIN = 128
N_WIN = N_SEG // WIN
ROWS = 232
NEG = -1e30


def _pick_block(n):
    for k in (2560, 2000, 1600, 1280, 1000, 800, 640, 512, 320, 256, 128, 64, 32, 16, 8):
        if n % k == 0:
            return k
    return n


def _body(nb, k_pts,
          s_ref, v_ref, pos_ref, bid_ref, bounds_ref,
          te_ref,
          a1v1_ref, a1v2_ref, a1swv_ref, a1sws_ref, a1gwt_ref, a1gb_ref, a1d_ref,
          a2v1_ref, a2swv_ref, a2sws_ref,
          n1v1_ref, n1v2_ref, n1swv_ref, n1sws_ref, n1gwt_ref, n1gb_ref, n1d_ref,
          n2v1_ref, n2v2_ref, n2swv_ref, n2sws_ref, n2gwt_ref, n2gb_ref,
          smat_ref, emat_ref,
          out_ref, m_ref, den_ref):
    i = pl.program_id(0)

    @pl.when(i == 0)
    def _init():
        out_ref[:] = jnp.zeros_like(out_ref)
        m_ref[:] = jnp.full_like(m_ref, NEG)
        den_ref[:] = jnp.zeros_like(den_ref)

    f32 = jnp.float32
    bf16 = jnp.bfloat16
    dot = functools.partial(jnp.dot, preferred_element_type=f32)

    def dot16(a, b):
        return jnp.dot(a.astype(bf16), b.astype(bf16),
                       preferred_element_type=f32)
    S = smat_ref[:]
    E = emat_ref[:]

    s0 = s_ref[:] + te_ref[:]
    v0 = v_ref[:]

    def gv(s, v, A1, A2, sWv, sWs, gWt, gb):
        vi = dot(v, A1)
        vn = jnp.sqrt(dot(vi * vi, S))
        os_ = dot(vn, sWv) + dot(s, sWs)
        ov = dot(vi, A2)
        gate = jax.nn.sigmoid(dot(os_, gWt) + gb)
        return os_, dot(gate, E) * ov

    def vlrelu(x, D):
        d = dot(x, D)
        dt = dot(x * d, S)
        dsq = dot(d * d, S)
        coef = jnp.where(dt >= 0.0, 0.0, dt / (dsq + 1e-9))
        return 0.2 * x + 0.8 * (x - dot(coef, E) * d)

    def lrelu(x):
        return jnp.where(x >= 0.0, x, 0.01 * x)

    sa, va = gv(s0, v0, a1v1_ref[:], a1v2_ref[:], a1swv_ref[:], a1sws_ref[:],
                a1gwt_ref[:], a1gb_ref[:])
    va = vlrelu(va, a1d_ref[:])
    sa = lrelu(sa)
    vi2 = dot(va, a2v1_ref[:])
    vn2 = jnp.sqrt(dot(vi2 * vi2, S))
    att_b = dot(vn2, a2swv_ref[:]) + dot(sa, a2sws_ref[:])

    sn, vn_ = gv(s0, v0, n1v1_ref[:], n1v2_ref[:], n1swv_ref[:], n1sws_ref[:],
                 n1gwt_ref[:], n1gb_ref[:])
    vn_ = vlrelu(vn_, n1d_ref[:])
    sn = lrelu(sn)
    hs, hv = gv(sn, vn_, n2v1_ref[:], n2v2_ref[:], n2swv_ref[:], n2sws_ref[:],
                n2gwt_ref[:], n2gb_ref[:])

    pos8 = jnp.concatenate(
        [pos_ref[:], jnp.zeros((k_pts, 5), dtype=f32)], axis=1)

    w2 = jnp.concatenate([jnp.full((1, WIN), 32.0, dtype=f32),
                          jnp.ones((1, WIN), dtype=f32)], axis=0)
    bid_b = dot(bid_ref[:], w2)
    bid_lo = bounds_ref[0, 0, 0]
    bid_hi = bounds_ref[0, 0, 1]
    lane = lax.broadcasted_iota(jnp.int32, (1, WIN), 1).astype(f32)

    for w in range(N_WIN):
        base = w * WIN

        @pl.when((bid_hi >= base) & (bid_lo < base + WIN))
        def _win(w=w, base=base):
            O = bid_b == (float(base) + lane)
            att_m = jnp.where(O, att_b, NEG)
            m_old = m_ref[w:w + 1, :]
            m_new = jnp.maximum(m_old, jnp.max(att_m, axis=0, keepdims=True))
            scale = jnp.exp(m_old - m_new)
            p = jnp.where(O, jnp.exp(att_m - m_new), 0.0)
            m_ref[w:w + 1, :] = m_new
            den_ref[w:w + 1, :] = (den_ref[w:w + 1, :] * scale
                                   + jnp.sum(p, axis=0, keepdims=True))
            cn = (((0,), (0,)), ((), ()))
            part_hs = lax.dot_general(hs, p, cn, preferred_element_type=f32)
            part_hv = lax.dot_general(hv, p, cn, preferred_element_type=f32)
            part_po = lax.dot_general(pos8, p, cn, preferred_element_type=f32)
            sl = slice(base, base + WIN)
            out_ref[0:128, sl] = out_ref[0:128, sl] * scale + part_hs
            out_ref[128:224, sl] = out_ref[128:224, sl] * scale + part_hv
            out_ref[224:232, sl] = out_ref[224:232, sl] * scale + part_po

    @pl.when(i == nb - 1)
    def _fin():
        den = den_ref[:]
        den_safe = jnp.where(den == 0.0, 1.0, den)
        for w in range(N_WIN):
            sl = slice(w * WIN, (w + 1) * WIN)
            out_ref[:, sl] = out_ref[:, sl] / den_safe[w:w + 1, :]


def kernel(h_att_sca, h_att_vec, pos_context, batch_id, t, params):
    n = h_att_sca.shape[0]
    hv_ch = h_att_vec.shape[1]
    k_pts = _pick_block(n)
    nb = n // k_pts
    f32 = jnp.float32

    eye3 = jnp.eye(3, dtype=f32)
    ones31 = jnp.ones((3, 1), dtype=f32)

    def kron3(W):
        return jnp.kron(W.T, eye3)

    p = params
    smat = jnp.kron(jnp.eye(hv_ch, dtype=f32), ones31)
    emat = smat.T

    te = p['time_embed'][t][None, :]
    v_flat = h_att_vec.reshape(n, -1)
    bidi = batch_id.astype(jnp.int32)
    bid2 = jnp.stack([(bidi >> 5).astype(f32),
                      (bidi & 31).astype(f32)], axis=1)
    bounds = jnp.stack([bidi[::k_pts], bidi[k_pts - 1::k_pts]],
                       axis=1).reshape(nb, 1, 2)

    hv_n = 3 * hv_ch
    args = [
        h_att_sca, v_flat, pos_context, bid2, bounds,
        te,
        kron3(p['a1_vW1']), kron3(p['a1_vW2']),
        p['a1_sW'][:, :hv_ch].T, p['a1_sW'][:, hv_ch:].T,
        p['a1_gW'].T, p['a1_gb'][None, :], kron3(p['a1_dW']),
        kron3(p['a2_vW1']),
        jnp.tile(p['a2_sW'][:, :hv_ch].T, (1, WIN)),
        jnp.tile(p['a2_sW'][:, hv_ch:].T, (1, WIN)),
        kron3(p['n1_vW1']), kron3(p['n1_vW2']),
        p['n1_sW'][:, :hv_ch].T, p['n1_sW'][:, hv_ch:].T,
        p['n1_gW'].T, p['n1_gb'][None, :], kron3(p['n1_dW']),
        kron3(p['n2_vW1']), kron3(p['n2_vW2']),
        p['n2_sW'][:, :hv_ch].T, p['n2_sW'][:, hv_ch:].T,
        p['n2_gW'].T, p['n2_gb'][None, :],
        smat, emat,
    ]

    def fixed(a):
        shape = a.shape
        return pl.BlockSpec(shape, lambda i: (0,) * len(shape))

    in_specs = [
        pl.BlockSpec((k_pts, 128), lambda i: (i, 0)),
        pl.BlockSpec((k_pts, hv_n), lambda i: (i, 0)),
        pl.BlockSpec((k_pts, 3), lambda i: (i, 0)),
        pl.BlockSpec((k_pts, 2), lambda i: (i, 0)),
        pl.BlockSpec((1, 1, 2), lambda i: (i, 0, 0), memory_space=pltpu.SMEM),
    ] + [fixed(a) for a in args[5:]]

    out = pl.pallas_call(
        functools.partial(_body, nb, k_pts),
        grid=(nb,),
        in_specs=in_specs,
        out_specs=pl.BlockSpec((ROWS, N_SEG), lambda i: (0, 0)),
        out_shape=jax.ShapeDtypeStruct((ROWS, N_SEG), f32),
        scratch_shapes=[
            pltpu.VMEM((N_WIN, WIN), f32),
            pltpu.VMEM((N_WIN, WIN), f32),
        ],
        compiler_params=pltpu.CompilerParams(
            dimension_semantics=("arbitrary",)),
    )(*args)

    feat = out[0:128, :].T
    vec = out[128:128 + hv_n, :].T.reshape(N_SEG, hv_ch, 3)
    pos = out[224:227, :].T
    return feat, vec, pos


# speedup vs baseline: 1.5582x; 1.5419x over previous
import functools

import jax
import jax.numpy as jnp
from jax import lax
from jax.experimental import pallas as pl
from jax.experimental.pallas import tpu as pltpu

N_SEG = 1024
WIN = 128
N_WIN = N_SEG // WIN
ROWS = 232
NEG = -1e30


def _pick_block(n):
    for k in (2560, 2048, 1536, 1280, 1024, 768, 640, 512, 384, 256, 128):
        if n % k == 0:
            return k
    return n


def _body(nb, k_pts,
          s_ref, v_ref, pos_ref, bid_ref, bounds_ref,
          te_ref,
          a1v1_ref, a1v2_ref, a1swv_ref, a1sws_ref, a1gwt_ref, a1gb_ref, a1d_ref,
          a2v1_ref, a2swv_ref, a2sws_ref,
          n1v1_ref, n1v2_ref, n1swv_ref, n1sws_ref, n1gwt_ref, n1gb_ref, n1d_ref,
          n2v1_ref, n2v2_ref, n2swv_ref, n2sws_ref, n2gwt_ref, n2gb_ref,
          sd_ref, bd_ref,
          out_ref, m_ref, den_ref):
    i = pl.program_id(0)

    @pl.when(i == 0)
    def _init():
        out_ref[:] = jnp.zeros_like(out_ref)
        m_ref[:] = jnp.full_like(m_ref, NEG)
        den_ref[:] = jnp.zeros_like(den_ref)

    f32 = jnp.float32
    dot = functools.partial(jnp.dot, preferred_element_type=f32)
    cn0 = (((0,), (0,)), ((), ()))

    def dot0(a, b):
        return lax.dot_general(a, b, cn0, preferred_element_type=f32)

    Sd = sd_ref[:]
    Bd = bd_ref[:]

    s0 = s_ref[:] + te_ref[:]
    v0 = v_ref[:]

    def gv(s, vT, A1, A2, sWv, sWs, gWt, gb):
        viT = dot(A1, vT)
        vn = jnp.sqrt(dot(Sd, viT * viT))
        os_ = dot0(vn, sWv) + dot(s, sWs)
        ovT = dot(A2, viT)
        gate = jax.nn.sigmoid(dot(os_, gWt) + gb)
        g96 = dot(Bd, gate.T)
        return os_, g96 * ovT

    def vlrelu(xT, D):
        dT = dot(D, xT)
        dt = dot(Sd, xT * dT)
        dsq = dot(Sd, dT * dT)
        coef = jnp.where(dt >= 0.0, 0.0, dt / (dsq + 1e-9))
        return 0.2 * xT + 0.8 * (xT - dot(Bd, coef) * dT)

    def lrelu(x):
        return jnp.where(x >= 0.0, x, 0.01 * x)

    sa, vaT = gv(s0, v0, a1v1_ref[:], a1v2_ref[:], a1swv_ref[:], a1sws_ref[:],
                 a1gwt_ref[:], a1gb_ref[:])
    vaT = vlrelu(vaT, a1d_ref[:])
    sa = lrelu(sa)
    vi2T = dot(a2v1_ref[:], vaT)
    vn2 = jnp.sqrt(dot(Sd, vi2T * vi2T))
    att_b = dot0(vn2, a2swv_ref[:]) + dot(sa, a2sws_ref[:])

    sn, vnT = gv(s0, v0, n1v1_ref[:], n1v2_ref[:], n1swv_ref[:], n1sws_ref[:],
                 n1gwt_ref[:], n1gb_ref[:])
    vnT = vlrelu(vnT, n1d_ref[:])
    sn = lrelu(sn)
    hs, hvT = gv(sn, vnT, n2v1_ref[:], n2v2_ref[:], n2swv_ref[:], n2sws_ref[:],
                 n2gwt_ref[:], n2gb_ref[:])

    posT = jnp.concatenate(
        [pos_ref[:], jnp.zeros((5, k_pts), dtype=f32)], axis=0)

    w2 = jnp.concatenate([jnp.full((1, WIN), 32.0, dtype=f32),
                          jnp.ones((1, WIN), dtype=f32)], axis=0)
    bid_b = dot0(bid_ref[:], w2)
    bid_lo = bounds_ref[0, 0, 0]
    bid_hi = bounds_ref[0, 0, 1]
    lane = lax.broadcasted_iota(jnp.int32, (1, WIN), 1).astype(f32)

    for w in range(N_WIN):
        base = w * WIN

        @pl.when((bid_hi >= base) & (bid_lo < base + WIN))
        def _win(w=w, base=base):
            O = bid_b == (float(base) + lane)
            att_m = jnp.where(O, att_b, NEG)
            m_old = m_ref[w:w + 1, :]
            m_new = jnp.maximum(m_old, jnp.max(att_m, axis=0, keepdims=True))
            scale = jnp.exp(m_old - m_new)
            p = jnp.where(O, jnp.exp(att_m - m_new), 0.0)
            m_ref[w:w + 1, :] = m_new
            den_ref[w:w + 1, :] = (den_ref[w:w + 1, :] * scale
                                   + jnp.sum(p, axis=0, keepdims=True))
            part_hs = dot0(hs, p)
            part_hv = dot(hvT, p)
            part_po = dot(posT, p)
            sl = slice(base, base + WIN)
            out_ref[0:128, sl] = out_ref[0:128, sl] * scale + part_hs
            out_ref[128:224, sl] = out_ref[128:224, sl] * scale + part_hv
            out_ref[224:232, sl] = out_ref[224:232, sl] * scale + part_po

    @pl.when(i == nb - 1)
    def _fin():
        den = den_ref[:]
        den_safe = jnp.where(den == 0.0, 1.0, den)
        for w in range(N_WIN):
            sl = slice(w * WIN, (w + 1) * WIN)
            out_ref[:, sl] = out_ref[:, sl] / den_safe[w:w + 1, :]


def kernel(h_att_sca, h_att_vec, pos_context, batch_id, t, params):
    n = h_att_sca.shape[0]
    hv_ch = h_att_vec.shape[1]
    k_pts = _pick_block(n)
    nb = n // k_pts
    f32 = jnp.float32

    eye3 = jnp.eye(3, dtype=f32)
    eyec = jnp.eye(hv_ch, dtype=f32)

    def kron3(W):
        return jnp.kron(eye3, W)

    p = params
    sd = jnp.kron(jnp.ones((1, 3), dtype=f32), eyec)
    bd = jnp.kron(jnp.ones((3, 1), dtype=f32), eyec)

    te = p['time_embed'][t][None, :]
    v_t = h_att_vec.transpose(2, 1, 0).reshape(3 * hv_ch, n)
    pos_t = pos_context.T
    bidi = batch_id.astype(jnp.int32)
    bid2 = jnp.stack([(bidi >> 5).astype(f32),
                      (bidi & 31).astype(f32)], axis=0)
    bounds = jnp.stack([bidi[::k_pts], bidi[k_pts - 1::k_pts]],
                       axis=1).reshape(nb, 1, 2)

    args = [
        h_att_sca, v_t, pos_t, bid2, bounds,
        te,
        kron3(p['a1_vW1']), kron3(p['a1_vW2']),
        p['a1_sW'][:, :hv_ch].T, p['a1_sW'][:, hv_ch:].T,
        p['a1_gW'].T, p['a1_gb'][None, :], kron3(p['a1_dW']),
        kron3(p['a2_vW1']),
        jnp.tile(p['a2_sW'][:, :hv_ch].T, (1, WIN)),
        jnp.tile(p['a2_sW'][:, hv_ch:].T, (1, WIN)),
        kron3(p['n1_vW1']), kron3(p['n1_vW2']),
        p['n1_sW'][:, :hv_ch].T, p['n1_sW'][:, hv_ch:].T,
        p['n1_gW'].T, p['n1_gb'][None, :], kron3(p['n1_dW']),
        kron3(p['n2_vW1']), kron3(p['n2_vW2']),
        p['n2_sW'][:, :hv_ch].T, p['n2_sW'][:, hv_ch:].T,
        p['n2_gW'].T, p['n2_gb'][None, :],
        sd, bd,
    ]

    def fixed(a):
        shape = a.shape
        return pl.BlockSpec(shape, lambda i: (0,) * len(shape))

    in_specs = [
        pl.BlockSpec((k_pts, 128), lambda i: (i, 0)),
        pl.BlockSpec((3 * hv_ch, k_pts), lambda i: (0, i)),
        pl.BlockSpec((3, k_pts), lambda i: (0, i)),
        pl.BlockSpec((2, k_pts), lambda i: (0, i)),
        pl.BlockSpec((1, 1, 2), lambda i: (i, 0, 0), memory_space=pltpu.SMEM),
    ] + [fixed(a) for a in args[5:]]

    out = pl.pallas_call(
        functools.partial(_body, nb, k_pts),
        grid=(nb,),
        in_specs=in_specs,
        out_specs=pl.BlockSpec((ROWS, N_SEG), lambda i: (0, 0)),
        out_shape=jax.ShapeDtypeStruct((ROWS, N_SEG), f32),
        scratch_shapes=[
            pltpu.VMEM((N_WIN, WIN), f32),
            pltpu.VMEM((N_WIN, WIN), f32),
        ],
        compiler_params=pltpu.CompilerParams(
            dimension_semantics=("arbitrary",)),
    )(*args)

    feat = out[0:128, :].T
    vec = out[128:128 + 3 * hv_ch, :].reshape(3, hv_ch, N_SEG).transpose(2, 1, 0)
    pos = out[224:227, :].T
    return feat, vec, pos


# speedup vs baseline: 1.5845x; 1.0169x over previous
import functools

import jax
import jax.numpy as jnp
from jax import lax
from jax.experimental import pallas as pl
from jax.experimental.pallas import tpu as pltpu

N_SEG = 1024
WIN = 128
N_WIN = N_SEG // WIN
ROWS = 232
NEG = -1e30


def _pick_block(n):
    for k in (2560, 2048, 1536, 1280, 1024, 768, 640, 512, 384, 256, 128):
        if n % k == 0:
            return k
    return n


def _body(nb, k_pts,
          s_ref, v_ref, pos_ref, bid_ref, bounds_ref,
          te_ref,
          a1w_ref, s1wv_ref, s1ws_ref, v1w2_ref, g1wt_ref, g1b_ref, d1w_ref,
          v2w1_ref, s2wv_ref, s2ws_ref, n2v2_ref, g2wt_ref, g2b_ref,
          sd2_ref, bd2_ref, bd_ref,
          out_ref, m_ref, den_ref):
    i = pl.program_id(0)

    @pl.when(i == 0)
    def _init():
        out_ref[:] = jnp.zeros_like(out_ref)
        m_ref[:] = jnp.full_like(m_ref, NEG)
        den_ref[:] = jnp.zeros_like(den_ref)

    f32 = jnp.float32
    dot = functools.partial(jnp.dot, preferred_element_type=f32)
    cn0 = (((0,), (0,)), ((), ()))

    def dot0(a, b):
        return lax.dot_general(a, b, cn0, preferred_element_type=f32)

    Sd2 = sd2_ref[:]
    Bd2 = bd2_ref[:]
    Bd = bd_ref[:]

    s0 = s_ref[:] + te_ref[:]
    v0 = v_ref[:]

    viT = dot(a1w_ref[:], v0)
    vn = jnp.sqrt(dot(Sd2, viT * viT))
    os_ = dot0(vn, s1wv_ref[:]) + dot(s0, s1ws_ref[:])
    ovT = dot(v1w2_ref[:], viT)
    gate = jax.nn.sigmoid(dot(os_, g1wt_ref[:]) + g1b_ref[:])
    v1 = dot(Bd2, gate.T) * ovT

    dT = dot(d1w_ref[:], v1)
    dt = dot(Sd2, v1 * dT)
    dsq = dot(Sd2, dT * dT)
    coef = jnp.where(dt >= 0.0, 0.0, dt / (dsq + 1e-9))
    v2 = 0.2 * v1 + 0.8 * (v1 - dot(Bd2, coef) * dT)
    s_act = jnp.where(os_ >= 0.0, os_, 0.01 * os_)

    vi2T = dot(v2w1_ref[:], v2)
    vn2 = jnp.sqrt(dot(Sd2, vi2T * vi2T))
    salin = dot0(vn2, s2wv_ref[:]) + dot(s_act, s2ws_ref[:])
    att_b = salin[:, 0:128]
    hs = salin[:, 128:256]
    ov2T = dot(n2v2_ref[:], vi2T[96:192, :])
    gate2 = jax.nn.sigmoid(dot(hs, g2wt_ref[:]) + g2b_ref[:])
    hvT = dot(Bd, gate2.T) * ov2T

    posT = jnp.concatenate(
        [pos_ref[:], jnp.zeros((5, k_pts), dtype=f32)], axis=0)

    w2 = jnp.concatenate([jnp.full((1, WIN), 32.0, dtype=f32),
                          jnp.ones((1, WIN), dtype=f32)], axis=0)
    bid_b = dot0(bid_ref[:], w2)
    bid_lo = bounds_ref[0, 0, 0]
    bid_hi = bounds_ref[0, 0, 1]
    lane = lax.broadcasted_iota(jnp.int32, (1, WIN), 1).astype(f32)

    for w in range(N_WIN):
        base = w * WIN

        @pl.when((bid_hi >= base) & (bid_lo < base + WIN))
        def _win(w=w, base=base):
            O = bid_b == (float(base) + lane)
            att_m = jnp.where(O, att_b, NEG)
            m_old = m_ref[w:w + 1, :]
            m_new = jnp.maximum(m_old, jnp.max(att_m, axis=0, keepdims=True))
            scale = jnp.exp(m_old - m_new)
            p = jnp.where(O, jnp.exp(att_m - m_new), 0.0)
            m_ref[w:w + 1, :] = m_new
            den_ref[w:w + 1, :] = (den_ref[w:w + 1, :] * scale
                                   + jnp.sum(p, axis=0, keepdims=True))
            part_hs = dot0(hs, p)
            part_hv = dot(hvT, p)
            part_po = dot(posT, p)
            sl = slice(base, base + WIN)
            out_ref[0:128, sl] = out_ref[0:128, sl] * scale + part_hs
            out_ref[128:224, sl] = out_ref[128:224, sl] * scale + part_hv
            out_ref[224:232, sl] = out_ref[224:232, sl] * scale + part_po

    @pl.when(i == nb - 1)
    def _fin():
        den = den_ref[:]
        den_safe = jnp.where(den == 0.0, 1.0, den)
        for w in range(N_WIN):
            sl = slice(w * WIN, (w + 1) * WIN)
            out_ref[:, sl] = out_ref[:, sl] / den_safe[w:w + 1, :]


def kernel(h_att_sca, h_att_vec, pos_context, batch_id, t, params):
    n = h_att_sca.shape[0]
    hv_ch = h_att_vec.shape[1]
    k_pts = _pick_block(n)
    nb = n // k_pts
    f32 = jnp.float32

    eye3 = jnp.eye(3, dtype=f32)
    eyec = jnp.eye(hv_ch, dtype=f32)

    def kron3(W):
        return jnp.kron(eye3, W)

    def bdiag(a, b):
        z1 = jnp.zeros((a.shape[0], b.shape[1]), dtype=f32)
        z2 = jnp.zeros((b.shape[0], a.shape[1]), dtype=f32)
        return jnp.block([[a, z1], [z2, b]])

    p = params
    sd = jnp.kron(jnp.ones((1, 3), dtype=f32), eyec)
    bd = jnp.kron(jnp.ones((3, 1), dtype=f32), eyec)

    te = p['time_embed'][t][None, :]
    v_t = h_att_vec.transpose(2, 1, 0).reshape(3 * hv_ch, n)
    pos_t = pos_context.T
    bidi = batch_id.astype(jnp.int32)
    bid2 = jnp.stack([(bidi >> 5).astype(f32),
                      (bidi & 31).astype(f32)], axis=0)
    bounds = jnp.stack([bidi[::k_pts], bidi[k_pts - 1::k_pts]],
                       axis=1).reshape(nb, 1, 2)

    args = [
        h_att_sca, v_t, pos_t, bid2, bounds,
        te,
        jnp.concatenate([kron3(p['a1_vW1']), kron3(p['n1_vW1'])], axis=0),
        bdiag(p['a1_sW'][:, :hv_ch].T, p['n1_sW'][:, :hv_ch].T),
        jnp.concatenate([p['a1_sW'][:, hv_ch:].T,
                         p['n1_sW'][:, hv_ch:].T], axis=1),
        bdiag(kron3(p['a1_vW2']), kron3(p['n1_vW2'])),
        bdiag(p['a1_gW'].T, p['n1_gW'].T),
        jnp.concatenate([p['a1_gb'], p['n1_gb']])[None, :],
        bdiag(kron3(p['a1_dW']), kron3(p['n1_dW'])),
        bdiag(kron3(p['a2_vW1']), kron3(p['n2_vW1'])),
        bdiag(jnp.tile(p['a2_sW'][:, :hv_ch].T, (1, WIN)),
              p['n2_sW'][:, :hv_ch].T),
        bdiag(jnp.tile(p['a2_sW'][:, hv_ch:].T, (1, WIN)),
              p['n2_sW'][:, hv_ch:].T),
        kron3(p['n2_vW2']),
        p['n2_gW'].T, p['n2_gb'][None, :],
        bdiag(sd, sd), bdiag(bd, bd), bd,
    ]

    def fixed(a):
        shape = a.shape
        return pl.BlockSpec(shape, lambda i: (0,) * len(shape))

    in_specs = [
        pl.BlockSpec((k_pts, 128), lambda i: (i, 0)),
        pl.BlockSpec((3 * hv_ch, k_pts), lambda i: (0, i)),
        pl.BlockSpec((3, k_pts), lambda i: (0, i)),
        pl.BlockSpec((2, k_pts), lambda i: (0, i)),
        pl.BlockSpec((1, 1, 2), lambda i: (i, 0, 0), memory_space=pltpu.SMEM),
    ] + [fixed(a) for a in args[5:]]

    out = pl.pallas_call(
        functools.partial(_body, nb, k_pts),
        grid=(nb,),
        in_specs=in_specs,
        out_specs=pl.BlockSpec((ROWS, N_SEG), lambda i: (0, 0)),
        out_shape=jax.ShapeDtypeStruct((ROWS, N_SEG), f32),
        scratch_shapes=[
            pltpu.VMEM((N_WIN, WIN), f32),
            pltpu.VMEM((N_WIN, WIN), f32),
        ],
        compiler_params=pltpu.CompilerParams(
            dimension_semantics=("arbitrary",)),
    )(*args)

    feat = out[0:128, :].T
    vec = out[128:128 + 3 * hv_ch, :].reshape(3, hv_ch, N_SEG).transpose(2, 1, 0)
    pos = out[224:227, :].T
    return feat, vec, pos


# speedup vs baseline: 1.6462x; 1.0390x over previous
import functools

import jax
import jax.numpy as jnp
from jax import lax
from jax.experimental import pallas as pl
from jax.experimental.pallas import tpu as pltpu

N_SEG = 1024
WIN = 128
N_WIN = N_SEG // WIN
ROWS = 232
NEG = -1e30


def _pick_block(n):
    for k in (2560, 2048, 1536, 1280, 1024, 768, 640, 512, 384, 256, 128):
        if n % k == 0:
            return k
    return n


def _body(nb, k_pts,
          s_ref, v_ref, pos_ref, bid_ref, bounds_ref,
          te_ref,
          a1w_ref, s1wv_ref, s1ws_ref, v1w2_ref, g1wt_ref, g1b_ref, d1w_ref,
          v2w1_ref, s2wv_ref, s2ws_ref, n2v2_ref, g2wt_ref, g2b_ref,
          sd2_ref, bd2_ref, bd2c_ref, bd_ref,
          out_ref, m_ref, den_ref):
    i = pl.program_id(0)

    @pl.when(i == 0)
    def _init():
        out_ref[:] = jnp.zeros_like(out_ref)
        m_ref[:] = jnp.full_like(m_ref, NEG)
        den_ref[:] = jnp.zeros_like(den_ref)

    f32 = jnp.float32
    dot = functools.partial(jnp.dot, preferred_element_type=f32)
    cn0 = (((0,), (0,)), ((), ()))

    def dot0(a, b):
        return lax.dot_general(a, b, cn0, preferred_element_type=f32)

    Sd2 = sd2_ref[:]
    Bd2 = bd2_ref[:]
    Bd = bd_ref[:]

    s0 = s_ref[:] + te_ref[:]
    v0 = v_ref[:]

    viT = dot(a1w_ref[:], v0)
    vn = jnp.sqrt(dot(Sd2, viT * viT))
    os_ = dot0(vn, s1wv_ref[:]) + dot(s0, s1ws_ref[:])
    ovT = dot(v1w2_ref[:], viT)
    gate = jax.nn.sigmoid(dot(os_, g1wt_ref[:]) + g1b_ref[:])
    v1 = dot(Bd2, gate.T) * ovT

    dT = dot(d1w_ref[:], v1)
    dt = dot(Sd2, v1 * dT)
    dsq = dot(Sd2, dT * dT)
    coef = jnp.where(dt >= 0.0, 0.0, dt / (dsq + 1e-9))
    v2 = v1 - dot(bd2c_ref[:], coef) * dT
    s_act = jnp.maximum(os_, 0.01 * os_)

    vi2T = dot(v2w1_ref[:], v2)
    vn2 = jnp.sqrt(dot(Sd2, vi2T * vi2T))
    salin = dot0(vn2, s2wv_ref[:]) + dot(s_act, s2ws_ref[:])
    att_b = salin[:, 0:128]
    hs = salin[:, 128:256]
    ov2T = dot(n2v2_ref[:], vi2T[96:192, :])
    gate2 = jax.nn.sigmoid(dot(hs, g2wt_ref[:]) + g2b_ref[:])
    hvT = dot(Bd, gate2.T) * ov2T
    vpT = jnp.concatenate(
        [hvT, pos_ref[:], jnp.zeros((5, k_pts), dtype=f32)], axis=0)

    w2 = jnp.concatenate([jnp.full((1, WIN), 32.0, dtype=f32),
                          jnp.ones((1, WIN), dtype=f32)], axis=0)
    bid_b = dot0(bid_ref[:], w2)
    bid_lo = bounds_ref[0, 0, 0]
    bid_hi = bounds_ref[0, 0, 1]
    lane = lax.broadcasted_iota(jnp.int32, (1, WIN), 1).astype(f32)

    for w in range(N_WIN):
        base = w * WIN

        @pl.when((bid_hi >= base) & (bid_lo < base + WIN))
        def _win(w=w, base=base):
            O = bid_b == (float(base) + lane)
            att_m = jnp.where(O, att_b, NEG)
            m_old = m_ref[w:w + 1, :]
            m_new = jnp.maximum(m_old, jnp.max(att_m, axis=0, keepdims=True))
            scale = jnp.exp(m_old - m_new)
            p = jnp.where(O, jnp.exp(att_m - m_new), 0.0)
            m_ref[w:w + 1, :] = m_new
            den_ref[w:w + 1, :] = (den_ref[w:w + 1, :] * scale
                                   + jnp.sum(p, axis=0, keepdims=True))
            part_hs = dot0(hs, p)
            part_vp = dot(vpT, p)
            sl = slice(base, base + WIN)
            out_ref[0:128, sl] = out_ref[0:128, sl] * scale + part_hs
            out_ref[128:232, sl] = out_ref[128:232, sl] * scale + part_vp

    @pl.when(i == nb - 1)
    def _fin():
        den = den_ref[:]
        den_safe = jnp.where(den == 0.0, 1.0, den)
        for w in range(N_WIN):
            sl = slice(w * WIN, (w + 1) * WIN)
            out_ref[:, sl] = out_ref[:, sl] / den_safe[w:w + 1, :]


def kernel(h_att_sca, h_att_vec, pos_context, batch_id, t, params):
    n = h_att_sca.shape[0]
    hv_ch = h_att_vec.shape[1]
    k_pts = _pick_block(n)
    nb = n // k_pts
    f32 = jnp.float32

    eye3 = jnp.eye(3, dtype=f32)
    eyec = jnp.eye(hv_ch, dtype=f32)

    def kron3(W):
        return jnp.kron(eye3, W)

    def bdiag(a, b):
        z1 = jnp.zeros((a.shape[0], b.shape[1]), dtype=f32)
        z2 = jnp.zeros((b.shape[0], a.shape[1]), dtype=f32)
        return jnp.block([[a, z1], [z2, b]])

    p = params
    sd = jnp.kron(jnp.ones((1, 3), dtype=f32), eyec)
    bd = jnp.kron(jnp.ones((3, 1), dtype=f32), eyec)

    te = p['time_embed'][t][None, :]
    v_t = h_att_vec.transpose(2, 1, 0).reshape(3 * hv_ch, n)
    pos_t = pos_context.T
    bidi = batch_id.astype(jnp.int32)
    bid2 = jnp.stack([(bidi >> 5).astype(f32),
                      (bidi & 31).astype(f32)], axis=0)
    bounds = jnp.stack([bidi[::k_pts], bidi[k_pts - 1::k_pts]],
                       axis=1).reshape(nb, 1, 2)

    args = [
        h_att_sca, v_t, pos_t, bid2, bounds,
        te,
        jnp.concatenate([kron3(p['a1_vW1']), kron3(p['n1_vW1'])], axis=0),
        bdiag(p['a1_sW'][:, :hv_ch].T, p['n1_sW'][:, :hv_ch].T),
        jnp.concatenate([p['a1_sW'][:, hv_ch:].T,
                         p['n1_sW'][:, hv_ch:].T], axis=1),
        bdiag(kron3(p['a1_vW2']), kron3(p['n1_vW2'])),
        bdiag(p['a1_gW'].T, p['n1_gW'].T),
        jnp.concatenate([p['a1_gb'], p['n1_gb']])[None, :],
        bdiag(kron3(p['a1_dW']), kron3(p['n1_dW'])),
        bdiag(kron3(p['a2_vW1']), kron3(p['n2_vW1'])),
        bdiag(jnp.tile(p['a2_sW'][:, :hv_ch].T, (1, WIN)),
              p['n2_sW'][:, :hv_ch].T),
        bdiag(jnp.tile(p['a2_sW'][:, hv_ch:].T, (1, WIN)),
              p['n2_sW'][:, hv_ch:].T),
        kron3(p['n2_vW2']),
        p['n2_gW'].T, p['n2_gb'][None, :],
        bdiag(sd, sd), bdiag(bd, bd), 0.8 * bdiag(bd, bd), bd,
    ]

    def fixed(a):
        shape = a.shape
        return pl.BlockSpec(shape, lambda i: (0,) * len(shape))

    in_specs = [
        pl.BlockSpec((k_pts, 128), lambda i: (i, 0)),
        pl.BlockSpec((3 * hv_ch, k_pts), lambda i: (0, i)),
        pl.BlockSpec((3, k_pts), lambda i: (0, i)),
        pl.BlockSpec((2, k_pts), lambda i: (0, i)),
        pl.BlockSpec((1, 1, 2), lambda i: (i, 0, 0), memory_space=pltpu.SMEM),
    ] + [fixed(a) for a in args[5:]]

    out = pl.pallas_call(
        functools.partial(_body, nb, k_pts),
        grid=(nb,),
        in_specs=in_specs,
        out_specs=pl.BlockSpec((ROWS, N_SEG), lambda i: (0, 0)),
        out_shape=jax.ShapeDtypeStruct((ROWS, N_SEG), f32),
        scratch_shapes=[
            pltpu.VMEM((N_WIN, WIN), f32),
            pltpu.VMEM((N_WIN, WIN), f32),
        ],
        compiler_params=pltpu.CompilerParams(
            dimension_semantics=("arbitrary",)),
    )(*args)

    feat = out[0:128, :].T
    vec = out[128:128 + 3 * hv_ch, :].reshape(3, hv_ch, N_SEG).transpose(2, 1, 0)
    pos = out[224:227, :].T
    return feat, vec, pos


# speedup vs baseline: 1.6747x; 1.0173x over previous
import functools

import jax
import jax.numpy as jnp
from jax import lax
from jax.experimental import pallas as pl
from jax.experimental.pallas import tpu as pltpu

N_SEG = 1024
WIN = 128
N_WIN = N_SEG // WIN
ROWS = 232
NEG = -1e30


def _pick_block(n):
    for k in (3200, 2560, 2048, 1536, 1280, 1024, 768, 640, 512, 384, 256, 128):
        if n % k == 0:
            return k
    return n


def _body(nb, k_pts,
          s_ref, v_ref, pos_ref, bid_ref, bounds_ref,
          te_ref,
          a1w_ref, s1wv_ref, s1ws_ref, v1w2_ref, g1wt_ref, g1b_ref, d1w_ref,
          v2w1_ref, s2wv_ref, s2ws_ref, n2v2_ref, g2wt_ref, g2b_ref,
          sd2_ref, bd2_ref, bd2c_ref, bd_ref,
          out_ref, m_ref, den_ref):
    i = pl.program_id(0)

    @pl.when(i == 0)
    def _init():
        out_ref[:] = jnp.zeros_like(out_ref)
        m_ref[:] = jnp.full_like(m_ref, NEG)
        den_ref[:] = jnp.zeros_like(den_ref)

    f32 = jnp.float32
    dot = functools.partial(jnp.dot, preferred_element_type=f32)
    cn0 = (((0,), (0,)), ((), ()))

    def dot0(a, b):
        return lax.dot_general(a, b, cn0, preferred_element_type=f32)

    Sd2 = sd2_ref[:]
    Bd2 = bd2_ref[:]
    Bd = bd_ref[:]

    s0 = s_ref[:] + te_ref[:]
    v0 = v_ref[:]

    viT = dot(a1w_ref[:], v0)
    vn = jnp.sqrt(dot(Sd2, viT * viT))
    os_ = dot0(vn, s1wv_ref[:]) + dot(s0, s1ws_ref[:])
    ovT = dot(v1w2_ref[:], viT)
    gate = jax.nn.sigmoid(dot(os_, g1wt_ref[:]) + g1b_ref[:])
    v1 = dot(Bd2, gate.T) * ovT

    dT = dot(d1w_ref[:], v1)
    dt = dot(Sd2, v1 * dT)
    dsq = dot(Sd2, dT * dT)
    coef = jnp.where(dt >= 0.0, 0.0, dt / (dsq + 1e-9))
    v2 = v1 - dot(bd2c_ref[:], coef) * dT
    s_act = jnp.maximum(os_, 0.01 * os_)

    vi2T = dot(v2w1_ref[:], v2)
    vn2 = jnp.sqrt(dot(Sd2, vi2T * vi2T))
    salin = dot0(vn2, s2wv_ref[:]) + dot(s_act, s2ws_ref[:])
    att_b = salin[:, 0:128]
    hs = salin[:, 128:256]
    ov2T = dot(n2v2_ref[:], vi2T[96:192, :])
    gate2 = jax.nn.sigmoid(dot(hs, g2wt_ref[:]) + g2b_ref[:])
    hvT = dot(Bd, gate2.T) * ov2T
    vpT = jnp.concatenate(
        [hvT, pos_ref[:], jnp.zeros((5, k_pts), dtype=f32)], axis=0)

    w2 = jnp.concatenate([jnp.full((1, WIN), 32.0, dtype=f32),
                          jnp.ones((1, WIN), dtype=f32)], axis=0)
    bid_b = dot0(bid_ref[:], w2)
    bid_lo = bounds_ref[0, 0, 0]
    bid_hi = bounds_ref[0, 0, 1]
    lane = lax.broadcasted_iota(jnp.int32, (1, WIN), 1).astype(f32)

    for w in range(N_WIN):
        base = w * WIN

        @pl.when((bid_hi >= base) & (bid_lo < base + WIN))
        def _win(w=w, base=base):
            O = bid_b == (float(base) + lane)
            att_m = jnp.where(O, att_b, NEG)
            m_old = m_ref[w:w + 1, :]
            m_new = jnp.maximum(m_old, jnp.max(att_m, axis=0, keepdims=True))
            scale = jnp.exp(m_old - m_new)
            p = jnp.where(O, jnp.exp(att_m - m_new), 0.0)
            m_ref[w:w + 1, :] = m_new
            den_ref[w:w + 1, :] = (den_ref[w:w + 1, :] * scale
                                   + jnp.sum(p, axis=0, keepdims=True))
            part_hs = dot0(hs, p)
            part_vp = dot(vpT, p)
            sl = slice(base, base + WIN)
            out_ref[0:128, sl] = out_ref[0:128, sl] * scale + part_hs
            out_ref[128:232, sl] = out_ref[128:232, sl] * scale + part_vp

    @pl.when(i == nb - 1)
    def _fin():
        den = den_ref[:]
        den_safe = jnp.where(den == 0.0, 1.0, den)
        for w in range(N_WIN):
            sl = slice(w * WIN, (w + 1) * WIN)
            out_ref[:, sl] = out_ref[:, sl] / den_safe[w:w + 1, :]


def kernel(h_att_sca, h_att_vec, pos_context, batch_id, t, params):
    n = h_att_sca.shape[0]
    hv_ch = h_att_vec.shape[1]
    k_pts = _pick_block(n)
    nb = n // k_pts
    f32 = jnp.float32

    eye3 = jnp.eye(3, dtype=f32)
    eyec = jnp.eye(hv_ch, dtype=f32)

    def kron3(W):
        return jnp.kron(eye3, W)

    def bdiag(a, b):
        z1 = jnp.zeros((a.shape[0], b.shape[1]), dtype=f32)
        z2 = jnp.zeros((b.shape[0], a.shape[1]), dtype=f32)
        return jnp.block([[a, z1], [z2, b]])

    p = params
    sd = jnp.kron(jnp.ones((1, 3), dtype=f32), eyec)
    bd = jnp.kron(jnp.ones((3, 1), dtype=f32), eyec)

    te = p['time_embed'][t][None, :]
    v_t = h_att_vec.transpose(2, 1, 0).reshape(3 * hv_ch, n)
    pos_t = pos_context.T
    bidi = batch_id.astype(jnp.int32)
    bid2 = jnp.stack([(bidi >> 5).astype(f32),
                      (bidi & 31).astype(f32)], axis=0)
    bounds = jnp.stack([bidi[::k_pts], bidi[k_pts - 1::k_pts]],
                       axis=1).reshape(nb, 1, 2)

    args = [
        h_att_sca, v_t, pos_t, bid2, bounds,
        te,
        jnp.concatenate([kron3(p['a1_vW1']), kron3(p['n1_vW1'])], axis=0),
        bdiag(p['a1_sW'][:, :hv_ch].T, p['n1_sW'][:, :hv_ch].T),
        jnp.concatenate([p['a1_sW'][:, hv_ch:].T,
                         p['n1_sW'][:, hv_ch:].T], axis=1),
        bdiag(kron3(p['a1_vW2']), kron3(p['n1_vW2'])),
        bdiag(p['a1_gW'].T, p['n1_gW'].T),
        jnp.concatenate([p['a1_gb'], p['n1_gb']])[None, :],
        bdiag(kron3(p['a1_dW']), kron3(p['n1_dW'])),
        bdiag(kron3(p['a2_vW1']), kron3(p['n2_vW1'])),
        bdiag(jnp.tile(p['a2_sW'][:, :hv_ch].T, (1, WIN)),
              p['n2_sW'][:, :hv_ch].T),
        bdiag(jnp.tile(p['a2_sW'][:, hv_ch:].T, (1, WIN)),
              p['n2_sW'][:, hv_ch:].T),
        kron3(p['n2_vW2']),
        p['n2_gW'].T, p['n2_gb'][None, :],
        bdiag(sd, sd), bdiag(bd, bd), 0.8 * bdiag(bd, bd), bd,
    ]

    def fixed(a):
        shape = a.shape
        return pl.BlockSpec(shape, lambda i: (0,) * len(shape))

    in_specs = [
        pl.BlockSpec((k_pts, 128), lambda i: (i, 0)),
        pl.BlockSpec((3 * hv_ch, k_pts), lambda i: (0, i)),
        pl.BlockSpec((3, k_pts), lambda i: (0, i)),
        pl.BlockSpec((2, k_pts), lambda i: (0, i)),
        pl.BlockSpec((1, 1, 2), lambda i: (i, 0, 0), memory_space=pltpu.SMEM),
    ] + [fixed(a) for a in args[5:]]

    out = pl.pallas_call(
        functools.partial(_body, nb, k_pts),
        grid=(nb,),
        in_specs=in_specs,
        out_specs=pl.BlockSpec((ROWS, N_SEG), lambda i: (0, 0)),
        out_shape=jax.ShapeDtypeStruct((ROWS, N_SEG), f32),
        scratch_shapes=[
            pltpu.VMEM((N_WIN, WIN), f32),
            pltpu.VMEM((N_WIN, WIN), f32),
        ],
        compiler_params=pltpu.CompilerParams(
            dimension_semantics=("arbitrary",)),
    )(*args)

    feat = out[0:128, :].T
    vec = out[128:128 + 3 * hv_ch, :].reshape(3, hv_ch, N_SEG).transpose(2, 1, 0)
    pos = out[224:227, :].T
    return feat, vec, pos
